# baseline (speedup 1.0000x reference)
import functools

import jax
import jax.numpy as jnp
from jax import lax
from jax.experimental import pallas as pl
from jax.experimental.pallas import tpu as pltpu
from jax.experimental.pallas import tpu_sc as plsc

_CHUNK = 128


@functools.lru_cache(maxsize=None)
def _make_sc_gather(num_rows, emb_dim, nw, nch):
    mesh = plsc.VectorSubcoreMesh(core_axis_name="c", subcore_axis_name="s")
    nc = mesh.num_cores

    @functools.partial(
        pl.kernel,
        mesh=mesh,
        out_type=jax.ShapeDtypeStruct((nw, nch, _CHUNK, emb_dim), jnp.float32),
        scratch_types=[
            pltpu.VMEM((nch, _CHUNK), jnp.int32),
            pltpu.VMEM((nch, _CHUNK, emb_dim), jnp.float32),
            pltpu.SemaphoreType.DMA,
        ],
        compiler_params=pltpu.CompilerParams(use_tc_tiling_on_sc=False),
    )
    def gather_k(table_hbm, idx_hbm, out_hbm, idx_v, rows_v, sem):
        wid = lax.axis_index("s") * nc + lax.axis_index("c")
        pltpu.sync_copy(idx_hbm.at[wid], idx_v)
        cps = [
            pltpu.async_copy(table_hbm.at[idx_v.at[j]], rows_v.at[j], sem)
            for j in range(nch)
        ]
        for cp in cps:
            cp.wait()
        pltpu.sync_copy(rows_v, out_hbm.at[wid])

    return gather_k


def _mlp_body(e_ref, w1_ref, b1_ref, w2_ref, b2_ref, wc_ref, bc_ref,
              feat_ref, logits_ref):
    h = jnp.maximum(
        jnp.dot(e_ref[...], w1_ref[...], preferred_element_type=jnp.float32)
        + b1_ref[...], 0.0)
    f = jnp.maximum(
        jnp.dot(h, w2_ref[...], preferred_element_type=jnp.float32)
        + b2_ref[...], 0.0)
    feat_ref[...] = f
    logits_ref[...] = (
        jnp.dot(f, wc_ref[...], preferred_element_type=jnp.float32)
        + bc_ref[...])


def _mlp_call(e, W1, b1, W2, b2, Wc_pad, bc_pad, bm, interpret=False):
    b, emb_dim = e.shape
    hidden = W1.shape[1]
    out_dim = W2.shape[1]
    ncls = Wc_pad.shape[1]
    nb = b // bm
    return pl.pallas_call(
        _mlp_body,
        grid=(nb,),
        in_specs=[
            pl.BlockSpec((bm, emb_dim), lambda i: (i, 0)),
            pl.BlockSpec((emb_dim, hidden), lambda i: (0, 0)),
            pl.BlockSpec((1, hidden), lambda i: (0, 0)),
            pl.BlockSpec((hidden, out_dim), lambda i: (0, 0)),
            pl.BlockSpec((1, out_dim), lambda i: (0, 0)),
            pl.BlockSpec((out_dim, ncls), lambda i: (0, 0)),
            pl.BlockSpec((1, ncls), lambda i: (0, 0)),
        ],
        out_specs=[
            pl.BlockSpec((bm, out_dim), lambda i: (i, 0)),
            pl.BlockSpec((bm, ncls), lambda i: (i, 0)),
        ],
        out_shape=[
            jax.ShapeDtypeStruct((b, out_dim), jnp.float32),
            jax.ShapeDtypeStruct((b, ncls), jnp.float32),
        ],
        interpret=interpret,
    )(e, W1, b1, W2, b2, Wc_pad, bc_pad)


def kernel(x, emb_table, W1, b1, W2, b2, Wc, bc):
    b = x.shape[0]
    num_rows, emb_dim = emb_table.shape
    nw = 32
    nch = b // (nw * _CHUNK)
    idx = x.astype(jnp.int32).reshape(nw, nch, _CHUNK)
    gather_k = _make_sc_gather(num_rows, emb_dim, nw, nch)
    e = gather_k(emb_table, idx).reshape(b, emb_dim)

    ncls = Wc.shape[1]
    ncls_pad = 128
    Wc_pad = jnp.pad(Wc, ((0, 0), (0, ncls_pad - ncls)))
    bc_pad = jnp.pad(bc, (0, ncls_pad - ncls)).reshape(1, ncls_pad)
    feat, logits_pad = _mlp_call(
        e, W1, b1.reshape(1, -1), W2, b2.reshape(1, -1), Wc_pad, bc_pad,
        bm=1024)
    return logits_pad[:, :ncls], feat
